# baseline (speedup 1.0000x reference)
import functools

import jax
import jax.numpy as jnp
from jax import lax
from jax.experimental import pallas as pl
from jax.experimental.pallas import tpu as pltpu
from jax.experimental.pallas import tpu_sc as plsc

B = 16384
F = 64
NC = 2
NS = 16
L = 16
NW = NC * NS
BPW = B // NW
NG = F * L
WROWS = 62500


def _make_mf_kernel():
    mesh = plsc.VectorSubcoreMesh(core_axis_name="c", subcore_axis_name="s")

    @functools.partial(
        pl.kernel,
        mesh=mesh,
        out_type=jax.ShapeDtypeStruct((B,), jnp.float32),
        compiler_params=pltpu.CompilerParams(needs_layout_passes=False,
                                             use_tc_tiling_on_sc=False),
        scratch_types=[
            pltpu.VMEM((BPW,), jnp.int32),
            pltpu.VMEM((BPW,), jnp.int32),
            pltpu.VMEM((BPW,), jnp.int32),
            pltpu.VMEM((BPW,), jnp.int32),
            pltpu.VMEM((BPW, L), jnp.float32),
            pltpu.VMEM((BPW, L), jnp.float32),
            pltpu.VMEM((NG,), jnp.int32),
            pltpu.VMEM((NG,), jnp.int32),
            pltpu.VMEM((NG, L), jnp.float32),
            pltpu.VMEM((NG, L), jnp.float32),
            pltpu.VMEM((BPW,), jnp.float32),
            pltpu.SemaphoreType.DMA,
        ],
    )
    def mf_kernel(user_hbm, movie_hbm, uf_hbm, mf_hbm, ub_hbm, mb_hbm,
                  out_hbm, uidx_v, midx_v, ubr_v, mbr_v, ubias_v, mbias_v,
                  fu_v, fm_v, du_v, dm_v, out_v, sem):
        wid = lax.axis_index("s") * NC + lax.axis_index("c")
        base = wid * BPW
        pltpu.sync_copy(user_hbm.at[pl.ds(base, BPW)], uidx_v)
        pltpu.sync_copy(movie_hbm.at[pl.ds(base, BPW)], midx_v)
        for t in range(BPW // L):
            ubr_v[pl.ds(t * L, L)] = jnp.right_shift(
                uidx_v[pl.ds(t * L, L)], 4)
            mbr_v[pl.ds(t * L, L)] = jnp.right_shift(
                midx_v[pl.ds(t * L, L)], 4)
        cb0 = pltpu.async_copy(ub_hbm.at[ubr_v], ubias_v, sem)
        cb1 = pltpu.async_copy(mb_hbm.at[mbr_v], mbias_v, sem)
        cb0.wait()
        cb1.wait()

        lanes = lax.iota(jnp.int32, L)
        lowmask = jnp.full((L,), L - 1, jnp.int32)

        def chunk(k, carry):
            r0 = k * L
            idu = uidx_v[pl.ds(r0, L)]
            idm = midx_v[pl.ds(r0, L)]
            bu = ubr_v[pl.ds(r0, L)]
            bm = mbr_v[pl.ds(r0, L)]
            for c in range(F):
                cc = jnp.full((L,), c * WROWS, jnp.int32)
                fu_v[pl.ds(c * L, L)] = bu + cc
                fm_v[pl.ds(c * L, L)] = bm + cc
            c0 = pltpu.async_copy(uf_hbm.at[fu_v], du_v, sem)
            c1 = pltpu.async_copy(mf_hbm.at[fm_v], dm_v, sem)
            c0.wait()
            c1.wait()
            ul = idu & lowmask
            ml = idm & lowmask
            rows = r0 + lanes
            acc = (plsc.load_gather(ubias_v, [rows, ul])
                   + plsc.load_gather(mbias_v, [rows, ml]))
            for c in range(F):
                rr = c * L + lanes
                uval = plsc.load_gather(du_v, [rr, ul])
                mval = plsc.load_gather(dm_v, [rr, ml])
                acc = acc + uval * mval
            out_v[pl.ds(r0, L)] = acc
            return carry

        lax.fori_loop(0, BPW // L, chunk, 0)
        pltpu.sync_copy(out_v, out_hbm.at[pl.ds(base, BPW)])

    return mf_kernel


_mf_kernel = _make_mf_kernel()


def kernel(user, movie, user_factors, movie_factors, user_biases,
           movie_biases):
    ufw = user_factors.T.reshape(F * WROWS * L // L, L)
    mfw = movie_factors.T.reshape(F * WROWS * L // L, L)
    ub = user_biases.reshape(WROWS, L)
    mb = movie_biases.reshape(WROWS, L)
    return _mf_kernel(user.astype(jnp.int32), movie.astype(jnp.int32),
                      ufw, mfw, ub, mb)


# speedup vs baseline: 9.8115x; 9.8115x over previous
import functools

import jax
import jax.numpy as jnp
from jax import lax
from jax.experimental import pallas as pl
from jax.experimental.pallas import tpu as pltpu
from jax.experimental.pallas import tpu_sc as plsc

B = 16384
F = 64
NC = 2
NS = 16
L = 16
NW = NC * NS
BPW = B // NW
C = 128
NCHUNK = BPW // C
NROWS = 1000000
NID = 1024
GRID = (NROWS + NID - 1) // NID
NPACK = GRID * (NID // 2)
NBPAD = 7816 * 128


def _relayout_pair(ufT, mfT):
    dn = (((0,), (0,)), ((), ()))

    def body(u_ref, m_ref, ou_ref, om_ref):
        r = jax.lax.broadcasted_iota(jnp.int32, (F, F), 0)
        c = jax.lax.broadcasted_iota(jnp.int32, (F, F), 1)
        eye = (r == c).astype(jnp.float32)

        def tr(x):
            return jax.lax.dot_general(x, eye, dimension_numbers=dn,
                                       preferred_element_type=jnp.float32)

        x = u_ref[...]
        ou_ref[:, 0:F] = tr(x[:, 0:NID // 2])
        ou_ref[:, F:2 * F] = tr(x[:, NID // 2:NID])
        y = m_ref[...]
        om_ref[:, 0:F] = tr(y[:, 0:NID // 2])
        om_ref[:, F:2 * F] = tr(y[:, NID // 2:NID])

    return pl.pallas_call(
        body,
        grid=(GRID,),
        in_specs=[pl.BlockSpec((F, NID), lambda k: (0, k)),
                  pl.BlockSpec((F, NID), lambda k: (0, k))],
        out_specs=[pl.BlockSpec((NID // 2, 2 * F), lambda k: (k, 0)),
                   pl.BlockSpec((NID // 2, 2 * F), lambda k: (k, 0))],
        out_shape=[jax.ShapeDtypeStruct((NPACK, 2 * F), jnp.float32),
                   jax.ShapeDtypeStruct((NPACK, 2 * F), jnp.float32)],
    )(ufT, mfT)


def _make_mf_kernel():
    mesh = plsc.VectorSubcoreMesh(core_axis_name="c", subcore_axis_name="s")

    @functools.partial(
        pl.kernel,
        mesh=mesh,
        out_type=jax.ShapeDtypeStruct((B,), jnp.float32),
        compiler_params=pltpu.CompilerParams(needs_layout_passes=False,
                                             use_tc_tiling_on_sc=True),
        scratch_types=[
            pltpu.VMEM((BPW,), jnp.int32),
            pltpu.VMEM((BPW,), jnp.int32),
            pltpu.VMEM((NCHUNK, C), jnp.int32),
            pltpu.VMEM((NCHUNK, C), jnp.int32),
            pltpu.VMEM((NCHUNK, C), jnp.int32),
            pltpu.VMEM((NCHUNK, C), jnp.int32),
            pltpu.VMEM((NCHUNK, C), jnp.int32),
            pltpu.VMEM((NCHUNK, C), jnp.int32),
            pltpu.VMEM((C, 2 * F), jnp.float32),
            pltpu.VMEM((C, 2 * F), jnp.float32),
            pltpu.VMEM((C, 128), jnp.float32),
            pltpu.VMEM((C, 128), jnp.float32),
            pltpu.VMEM((BPW,), jnp.float32),
            pltpu.SemaphoreType.DMA,
        ],
    )
    def mf_kernel(user_hbm, movie_hbm, uf_hbm, mf_hbm, ub_hbm, mb_hbm,
                  out_hbm, uidx_v, midx_v, udma_v, mdma_v, ubd_v, mbd_v,
                  ucs_v, mcs_v, ubuf_v, mbuf_v, ubb_v, mbb_v, out_v, sem):
        wid = lax.axis_index("s") * NC + lax.axis_index("c")
        base = wid * BPW
        pltpu.sync_copy(user_hbm.at[pl.ds(base, BPW)], uidx_v)
        pltpu.sync_copy(movie_hbm.at[pl.ds(base, BPW)], midx_v)
        m511 = jnp.full((L,), 511, jnp.int32)
        m1 = jnp.full((L,), 1, jnp.int32)
        c64v = jnp.full((L,), 64, jnp.int32)

        def rowcol(x):
            row = jnp.left_shift(jnp.right_shift(x, 10), 9) + (x & m511)
            col = (jnp.right_shift(x, 9) & m1) * c64v
            return row, col

        for k in range(NCHUNK):
            for j in range(C // L):
                u = uidx_v[pl.ds(k * C + j * L, L)]
                m = midx_v[pl.ds(k * C + j * L, L)]
                urow, ucol = rowcol(u)
                mrow, mcol = rowcol(m)
                udma_v[k, pl.ds(j * L, L)] = urow
                mdma_v[k, pl.ds(j * L, L)] = mrow
                ucs_v[k, pl.ds(j * L, L)] = ucol
                mcs_v[k, pl.ds(j * L, L)] = mcol
                ubd_v[k, pl.ds(j * L, L)] = jnp.right_shift(u, 7)
                mbd_v[k, pl.ds(j * L, L)] = jnp.right_shift(m, 7)

        lanes = lax.iota(jnp.int32, L)
        m127 = jnp.full((L,), 127, jnp.int32)

        for k in range(NCHUNK):
            c0 = pltpu.async_copy(uf_hbm.at[udma_v.at[k]], ubuf_v, sem)
            c1 = pltpu.async_copy(mf_hbm.at[mdma_v.at[k]], mbuf_v, sem)
            c2 = pltpu.async_copy(ub_hbm.at[ubd_v.at[k]], ubb_v, sem)
            c3 = pltpu.async_copy(mb_hbm.at[mbd_v.at[k]], mbb_v, sem)
            c0.wait()
            c1.wait()
            c2.wait()
            c3.wait()
            for s in range(C // L):
                rows = s * L + lanes
                idu = uidx_v[pl.ds(k * C + s * L, L)]
                idm = midx_v[pl.ds(k * C + s * L, L)]
                ucol = ucs_v[k, pl.ds(s * L, L)]
                mcol = mcs_v[k, pl.ds(s * L, L)]
                acc = (plsc.load_gather(ubb_v, [rows, idu & m127])
                       + plsc.load_gather(mbb_v, [rows, idm & m127]))
                for c in range(F):
                    cc = jnp.full((L,), c, jnp.int32)
                    uval = plsc.load_gather(ubuf_v, [rows, ucol + cc])
                    mval = plsc.load_gather(mbuf_v, [rows, mcol + cc])
                    acc = acc + uval * mval
                out_v[pl.ds(k * C + s * L, L)] = acc

        pltpu.sync_copy(out_v, out_hbm.at[pl.ds(base, BPW)])

    return mf_kernel


_mf_kernel = _make_mf_kernel()


def kernel(user, movie, user_factors, movie_factors, user_biases,
           movie_biases):
    ufv, mfv = _relayout_pair(user_factors.T, movie_factors.T)
    ubp = jnp.pad(user_biases.reshape(-1), (0, NBPAD - NROWS))
    mbp = jnp.pad(movie_biases.reshape(-1), (0, NBPAD - NROWS))
    return _mf_kernel(user.astype(jnp.int32), movie.astype(jnp.int32),
                      ufv, mfv, ubp.reshape(7816, 128),
                      mbp.reshape(7816, 128))


# speedup vs baseline: 9.8921x; 1.0082x over previous
import functools

import jax
import jax.numpy as jnp
from jax import lax
from jax.experimental import pallas as pl
from jax.experimental.pallas import tpu as pltpu
from jax.experimental.pallas import tpu_sc as plsc

B = 16384
F = 64
NC = 2
NS = 16
L = 16
NW = NC * NS
BPW = B // NW
C = 128
NCHUNK = BPW // C
NROWS = 1000000
NID = 1024
GRID = (NROWS + NID - 1) // NID
NPACK = GRID * (NID // 2)
NBPAD = 7816 * 128


def _relayout_pair(ufT, mfT):
    def body(u_ref, m_ref, ou_ref, om_ref):
        x = u_ref[...]
        ou_ref[:, 0:F] = x[:, 0:NID // 2].T
        ou_ref[:, F:2 * F] = x[:, NID // 2:NID].T
        y = m_ref[...]
        om_ref[:, 0:F] = y[:, 0:NID // 2].T
        om_ref[:, F:2 * F] = y[:, NID // 2:NID].T

    return pl.pallas_call(
        body,
        grid=(GRID,),
        in_specs=[pl.BlockSpec((F, NID), lambda k: (0, k)),
                  pl.BlockSpec((F, NID), lambda k: (0, k))],
        out_specs=[pl.BlockSpec((NID // 2, 2 * F), lambda k: (k, 0)),
                   pl.BlockSpec((NID // 2, 2 * F), lambda k: (k, 0))],
        out_shape=[jax.ShapeDtypeStruct((NPACK, 2 * F), jnp.float32),
                   jax.ShapeDtypeStruct((NPACK, 2 * F), jnp.float32)],
    )(ufT, mfT)


def _make_mf_kernel():
    mesh = plsc.VectorSubcoreMesh(core_axis_name="c", subcore_axis_name="s")

    @functools.partial(
        pl.kernel,
        mesh=mesh,
        out_type=jax.ShapeDtypeStruct((B,), jnp.float32),
        compiler_params=pltpu.CompilerParams(needs_layout_passes=False,
                                             use_tc_tiling_on_sc=True),
        scratch_types=[
            pltpu.VMEM((BPW,), jnp.int32),
            pltpu.VMEM((BPW,), jnp.int32),
            pltpu.VMEM((NCHUNK, C), jnp.int32),
            pltpu.VMEM((NCHUNK, C), jnp.int32),
            pltpu.VMEM((NCHUNK, C), jnp.int32),
            pltpu.VMEM((NCHUNK, C), jnp.int32),
            pltpu.VMEM((NCHUNK, C), jnp.int32),
            pltpu.VMEM((NCHUNK, C), jnp.int32),
            pltpu.VMEM((C, 2 * F), jnp.float32),
            pltpu.VMEM((C, 2 * F), jnp.float32),
            pltpu.VMEM((C, 128), jnp.float32),
            pltpu.VMEM((C, 128), jnp.float32),
            pltpu.VMEM((BPW,), jnp.float32),
            pltpu.SemaphoreType.DMA,
        ],
    )
    def mf_kernel(user_hbm, movie_hbm, uf_hbm, mf_hbm, ub_hbm, mb_hbm,
                  out_hbm, uidx_v, midx_v, udma_v, mdma_v, ubd_v, mbd_v,
                  ucs_v, mcs_v, ubuf_v, mbuf_v, ubb_v, mbb_v, out_v, sem):
        wid = lax.axis_index("s") * NC + lax.axis_index("c")
        base = wid * BPW
        pltpu.sync_copy(user_hbm.at[pl.ds(base, BPW)], uidx_v)
        pltpu.sync_copy(movie_hbm.at[pl.ds(base, BPW)], midx_v)
        m511 = jnp.full((L,), 511, jnp.int32)
        m1 = jnp.full((L,), 1, jnp.int32)
        c64v = jnp.full((L,), 64, jnp.int32)

        def rowcol(x):
            row = jnp.left_shift(jnp.right_shift(x, 10), 9) + (x & m511)
            col = (jnp.right_shift(x, 9) & m1) * c64v
            return row, col

        for k in range(NCHUNK):
            for j in range(C // L):
                u = uidx_v[pl.ds(k * C + j * L, L)]
                m = midx_v[pl.ds(k * C + j * L, L)]
                urow, ucol = rowcol(u)
                mrow, mcol = rowcol(m)
                udma_v[k, pl.ds(j * L, L)] = urow
                mdma_v[k, pl.ds(j * L, L)] = mrow
                ucs_v[k, pl.ds(j * L, L)] = ucol
                mcs_v[k, pl.ds(j * L, L)] = mcol
                ubd_v[k, pl.ds(j * L, L)] = jnp.right_shift(u, 7)
                mbd_v[k, pl.ds(j * L, L)] = jnp.right_shift(m, 7)

        lanes = lax.iota(jnp.int32, L)
        m127 = jnp.full((L,), 127, jnp.int32)

        for k in range(NCHUNK):
            c0 = pltpu.async_copy(uf_hbm.at[udma_v.at[k]], ubuf_v, sem)
            c1 = pltpu.async_copy(mf_hbm.at[mdma_v.at[k]], mbuf_v, sem)
            c2 = pltpu.async_copy(ub_hbm.at[ubd_v.at[k]], ubb_v, sem)
            c3 = pltpu.async_copy(mb_hbm.at[mbd_v.at[k]], mbb_v, sem)
            c0.wait()
            c1.wait()
            c2.wait()
            c3.wait()
            for s in range(C // L):
                rows = s * L + lanes
                idu = uidx_v[pl.ds(k * C + s * L, L)]
                idm = midx_v[pl.ds(k * C + s * L, L)]
                ucol = ucs_v[k, pl.ds(s * L, L)]
                mcol = mcs_v[k, pl.ds(s * L, L)]
                acc = (plsc.load_gather(ubb_v, [rows, idu & m127])
                       + plsc.load_gather(mbb_v, [rows, idm & m127]))
                for c in range(F):
                    cc = jnp.full((L,), c, jnp.int32)
                    uval = plsc.load_gather(ubuf_v, [rows, ucol + cc])
                    mval = plsc.load_gather(mbuf_v, [rows, mcol + cc])
                    acc = acc + uval * mval
                out_v[pl.ds(k * C + s * L, L)] = acc

        pltpu.sync_copy(out_v, out_hbm.at[pl.ds(base, BPW)])

    return mf_kernel


_mf_kernel = _make_mf_kernel()


def kernel(user, movie, user_factors, movie_factors, user_biases,
           movie_biases):
    ufv, mfv = _relayout_pair(user_factors.T, movie_factors.T)
    ubp = jnp.pad(user_biases.reshape(-1), (0, NBPAD - NROWS))
    mbp = jnp.pad(movie_biases.reshape(-1), (0, NBPAD - NROWS))
    return _mf_kernel(user.astype(jnp.int32), movie.astype(jnp.int32),
                      ufv, mfv, ubp.reshape(7816, 128),
                      mbp.reshape(7816, 128))
